# baseline (speedup 1.0000x reference)
import jax
import jax.numpy as jnp
from jax.experimental import pallas as pl
from jax.experimental.pallas import tpu as pltpu

N = 10000
NFEAT = 128
H1 = 64
H2 = 32

ROWS_BLK = 512
NB = -(-N // ROWS_BLK)

_PARAMS = pltpu.CompilerParams(
    dimension_semantics=("arbitrary",),
    vmem_limit_bytes=64 * 1024 * 1024,
)


def _gcn_body(x_ref, adj_ref, w1_ref, b1_ref, w2_ref, b2_ref, o_ref, s1_ref, s2_ref):
    g = pl.program_id(0)

    @pl.when(g == 0)
    def _():
        s1_ref[...] = jnp.dot(
            x_ref[...], w1_ref[...], preferred_element_type=jnp.float32
        )

    @pl.when((g >= 1) & (g <= NB))
    def _():
        i = NB - g
        h = jnp.dot(adj_ref[...], s1_ref[...], preferred_element_type=jnp.float32)
        h = jnp.maximum(h + b1_ref[...], 0.0)
        s2_ref[pl.ds(i * ROWS_BLK, ROWS_BLK), :] = jnp.dot(
            h, w2_ref[...], preferred_element_type=jnp.float32
        )

    @pl.when(g > NB)
    def _():
        o_ref[...] = (
            jnp.dot(adj_ref[...], s2_ref[0:N, :], preferred_element_type=jnp.float32)
            + b2_ref[...]
        )


def _adj_index(g):
    p1 = NB - g
    p2 = g - 1 - NB
    return (jnp.where(g == 0, NB - 1, jnp.where(g <= NB, p1, p2)), 0)


def _out_index(g):
    return (jnp.maximum(g - 1 - NB, 0), 0)


@jax.jit
def _gcn(x, adj, W1, b1, W2, b2):
    b1r = b1.reshape(1, H1)
    b2r = b2.reshape(1, H2)

    out = pl.pallas_call(
        _gcn_body,
        grid=(1 + 2 * NB,),
        in_specs=[
            pl.BlockSpec((N, NFEAT), lambda g: (0, 0)),
            pl.BlockSpec((ROWS_BLK, N), _adj_index),
            pl.BlockSpec((NFEAT, H1), lambda g: (0, 0)),
            pl.BlockSpec((1, H1), lambda g: (0, 0)),
            pl.BlockSpec((H1, H2), lambda g: (0, 0)),
            pl.BlockSpec((1, H2), lambda g: (0, 0)),
        ],
        out_specs=pl.BlockSpec((ROWS_BLK, H2), _out_index),
        out_shape=jax.ShapeDtypeStruct((N, H2), jnp.float32),
        scratch_shapes=[
            pltpu.VMEM((N, H1), jnp.float32),
            pltpu.VMEM((NB * ROWS_BLK, H2), jnp.float32),
        ],
        compiler_params=_PARAMS,
    )(x, adj, W1, b1r, W2, b2r)

    return out


def kernel(x, adj, W1, b1, W2, b2):
    return _gcn(x, adj, W1, b1, W2, b2)


# speedup vs baseline: 1.0172x; 1.0172x over previous
import jax
import jax.numpy as jnp
from jax.experimental import pallas as pl
from jax.experimental.pallas import tpu as pltpu

N = 10000
NFEAT = 128
H1 = 64
H2 = 32

ROWS_BLK = 400
NB = N // ROWS_BLK
HR = ROWS_BLK // 2

_PARAMS = pltpu.CompilerParams(
    dimension_semantics=("arbitrary",),
    vmem_limit_bytes=64 * 1024 * 1024,
)


def _gcn_body(
    x_ref, adjt_ref, adjb_ref, w1_ref, b1_ref, w2_ref, b2_ref, o_ref, s1_ref, s2_ref
):
    g = pl.program_id(0)

    @pl.when(g == 0)
    def _():
        s1_ref[...] = jnp.dot(
            x_ref[...], w1_ref[...], preferred_element_type=jnp.float32
        )

    @pl.when((g >= 1) & (g <= NB))
    def _():
        i = NB - g
        ht = jnp.dot(adjt_ref[...], s1_ref[...], preferred_element_type=jnp.float32)
        hb = jnp.dot(adjb_ref[...], s1_ref[...], preferred_element_type=jnp.float32)
        ht = jnp.maximum(ht + b1_ref[...], 0.0)
        hb = jnp.maximum(hb + b1_ref[...], 0.0)
        s2_ref[pl.ds(i * ROWS_BLK, HR), :] = jnp.dot(
            ht, w2_ref[...], preferred_element_type=jnp.float32
        )
        s2_ref[pl.ds(i * ROWS_BLK + HR, HR), :] = jnp.dot(
            hb, w2_ref[...], preferred_element_type=jnp.float32
        )

    @pl.when(g > NB)
    def _():
        o_ref[0:HR, :] = (
            jnp.dot(adjt_ref[...], s2_ref[...], preferred_element_type=jnp.float32)
            + b2_ref[...]
        )
        o_ref[HR : 2 * HR, :] = (
            jnp.dot(adjb_ref[...], s2_ref[...], preferred_element_type=jnp.float32)
            + b2_ref[...]
        )


def _adj_row(g):
    p1 = NB - g
    p2 = g - 1 - NB
    return jnp.where(g == 0, NB - 1, jnp.where(g <= NB, p1, p2))


def _adjt_index(g):
    return (2 * _adj_row(g), 0)


def _adjb_index(g):
    return (2 * _adj_row(g) + 1, 0)


def _out_index(g):
    return (jnp.maximum(g - 1 - NB, 0), 0)


@jax.jit
def _gcn(x, adj, W1, b1, W2, b2):
    b1r = b1.reshape(1, H1)
    b2r = b2.reshape(1, H2)

    out = pl.pallas_call(
        _gcn_body,
        grid=(1 + 2 * NB,),
        in_specs=[
            pl.BlockSpec((N, NFEAT), lambda g: (0, 0)),
            pl.BlockSpec((HR, N), _adjt_index),
            pl.BlockSpec((HR, N), _adjb_index),
            pl.BlockSpec((NFEAT, H1), lambda g: (0, 0)),
            pl.BlockSpec((1, H1), lambda g: (0, 0)),
            pl.BlockSpec((H1, H2), lambda g: (0, 0)),
            pl.BlockSpec((1, H2), lambda g: (0, 0)),
        ],
        out_specs=pl.BlockSpec((ROWS_BLK, H2), _out_index),
        out_shape=jax.ShapeDtypeStruct((N, H2), jnp.float32),
        scratch_shapes=[
            pltpu.VMEM((N, H1), jnp.float32),
            pltpu.VMEM((N, H2), jnp.float32),
        ],
        compiler_params=_PARAMS,
    )(x, adj, adj, W1, b1r, W2, b2r)

    return out


def kernel(x, adj, W1, b1, W2, b2):
    return _gcn(x, adj, W1, b1, W2, b2)
